# baseline (speedup 1.0000x reference)
import functools

import jax
import jax.numpy as jnp
from jax import lax
from jax.experimental import pallas as pl
from jax.experimental.pallas import tpu as pltpu
from jax.experimental.pallas import tpu_sc as plsc


B, N, M = 8, 2048, 2048
NC, NS, L = 2, 16, 16
NW = NC * NS
GPB = NW // B
CH = N // GPB
JB = M // L
ITILE = 4

_mesh = plsc.VectorSubcoreMesh(core_axis_name="c", subcore_axis_name="s")

_GDN = lax.GatherDimensionNumbers(
    offset_dims=(), collapsed_slice_dims=(0,), start_index_map=(0,))


def _perm(v, idx):
    return lax.gather(v, idx[:, None], dimension_numbers=_GDN,
                      slice_sizes=(1,),
                      mode=lax.GatherScatterMode.PROMISE_IN_BOUNDS)


def _xlane_reduce(v, op):
    lane = lax.iota(jnp.int32, L)
    for sh in (8, 4, 2, 1):
        v = op(v, _perm(v, lane ^ sh))
    return v[0]


@functools.partial(
    pl.kernel,
    out_type=[
        jax.ShapeDtypeStruct((NW, M), jnp.float32),
        jax.ShapeDtypeStruct((NW, L), jnp.float32),
    ],
    mesh=_mesh,
    scratch_types=[
        pltpu.VMEM((CH,), jnp.float32),
        pltpu.VMEM((CH,), jnp.float32),
        pltpu.VMEM((CH,), jnp.float32),
        pltpu.VMEM((M,), jnp.float32),
        pltpu.VMEM((M,), jnp.float32),
        pltpu.VMEM((M,), jnp.float32),
        pltpu.VMEM((M,), jnp.float32),
        pltpu.VMEM((L,), jnp.float32),
    ],
)
def _stage1(gx_h, gy_h, gz_h, px_h, py_h, pz_h, colmin_h, rowsum_h,
            gx, gy, gz, px, py, pz, colmin, rs_v):
    wid = lax.axis_index("c") * NS + lax.axis_index("s")
    b = wid // GPB
    chunk = wid % GPB
    g0 = chunk * CH

    pltpu.sync_copy(gx_h.at[b, pl.ds(g0, CH)], gx)
    pltpu.sync_copy(gy_h.at[b, pl.ds(g0, CH)], gy)
    pltpu.sync_copy(gz_h.at[b, pl.ds(g0, CH)], gz)
    pltpu.sync_copy(px_h.at[b], px)
    pltpu.sync_copy(py_h.at[b], py)
    pltpu.sync_copy(pz_h.at[b], pz)

    inf_v = jnp.full((L,), jnp.inf, jnp.float32)

    def init_body(j, carry):
        colmin[pl.ds(j * L, L)] = inf_v
        return carry

    lax.fori_loop(0, JB, init_body, jnp.int32(0))

    def group_body(it, rowsum):
        base = it * L
        gxv = gx[pl.ds(base, L)]
        gyv = gy[pl.ds(base, L)]
        gzv = gz[pl.ds(base, L)]
        for kk in range(L // ITILE):
            gs = [(gxv[kk * ITILE + k], gyv[kk * ITILE + k],
                   gzv[kk * ITILE + k]) for k in range(ITILE)]

            def jb_body(j, rms):
                o = j * L
                pxv = px[pl.ds(o, L)]
                pyv = py[pl.ds(o, L)]
                pzv = pz[pl.ds(o, L)]
                cm = colmin[pl.ds(o, L)]
                new_rms = []
                for k in range(ITILE):
                    gxk, gyk, gzk = gs[k]
                    dx = pxv - gxk
                    dy = pyv - gyk
                    dz = pzv - gzk
                    d2 = dx * dx + dy * dy + dz * dz
                    new_rms.append(jnp.minimum(rms[k], d2))
                    cm = jnp.minimum(cm, d2)
                colmin[pl.ds(o, L)] = cm
                return tuple(new_rms)

            rms = lax.fori_loop(0, JB, jb_body,
                                tuple(inf_v for _ in range(ITILE)))
            for k in range(ITILE):
                rowsum = rowsum + _xlane_reduce(rms[k], jnp.minimum)
        return rowsum

    rowsum = lax.fori_loop(0, CH // L, group_body, jnp.float32(0))

    pltpu.sync_copy(colmin, colmin_h.at[wid])
    lane = lax.iota(jnp.int32, L)
    rs_v[...] = jnp.where(lane == 0, rowsum, jnp.float32(0))
    pltpu.sync_copy(rs_v, rowsum_h.at[wid])


@functools.partial(
    pl.kernel,
    out_type=jax.ShapeDtypeStruct((L,), jnp.float32),
    mesh=_mesh,
    scratch_types=[
        pltpu.VMEM((NW, M), jnp.float32),
        pltpu.VMEM((NW, L), jnp.float32),
        pltpu.VMEM((L,), jnp.float32),
    ],
)
def _stage2(colmin_h, rowsum_h, out_h, cm_v, rs_v, o_v):
    wid = lax.axis_index("c") * NS + lax.axis_index("s")

    @pl.when(wid == 0)
    def _():
        pltpu.sync_copy(colmin_h, cm_v)
        pltpu.sync_copy(rowsum_h, rs_v)

        def col_body(t, acc):
            bb = t // JB
            j = t % JB
            o = j * L
            w0 = bb * GPB
            m = cm_v[w0, pl.ds(o, L)]
            m = jnp.minimum(m, cm_v[w0 + 1, pl.ds(o, L)])
            m = jnp.minimum(m, cm_v[w0 + 2, pl.ds(o, L)])
            m = jnp.minimum(m, cm_v[w0 + 3, pl.ds(o, L)])
            return acc + m

        col_acc = lax.fori_loop(0, B * JB, col_body,
                                jnp.zeros((L,), jnp.float32))

        def row_body(w, acc):
            return acc + rs_v[w]

        row_acc = lax.fori_loop(0, NW, row_body, jnp.zeros((L,), jnp.float32))

        total = (_xlane_reduce(col_acc, jnp.add) +
                 _xlane_reduce(row_acc, jnp.add)) * jnp.float32(1.0 / (B * N))
        lane = lax.iota(jnp.int32, L)
        o_v[...] = jnp.where(lane == 0, total, jnp.float32(0))
        pltpu.sync_copy(o_v, out_h)


def kernel(gt, pred):
    gx = jnp.asarray(gt[:, :, 0])
    gy = jnp.asarray(gt[:, :, 1])
    gz = jnp.asarray(gt[:, :, 2])
    px = jnp.asarray(pred[:, :, 0])
    py = jnp.asarray(pred[:, :, 1])
    pz = jnp.asarray(pred[:, :, 2])
    colmin, rowsum = _stage1(gx, gy, gz, px, py, pz)
    out = _stage2(colmin, rowsum)
    return out[0]
